# baseline (speedup 1.0000x reference)
import functools

import jax
import jax.numpy as jnp
from jax import lax
from jax.experimental import pallas as pl
from jax.experimental.pallas import tpu as pltpu
from jax.experimental.pallas import tpu_sc as plsc

NC = 2
NS = 16
NW = NC * NS
L = 16
CHUNK = 128

MAX_NORM = 1.0
EPS = 1e-7


def _rsqrt(x):
    i = plsc.bitcast(x, jnp.int32)
    i = jnp.int32(0x5F3759DF) - (i >> 1)
    y = plsc.bitcast(i, jnp.float32)
    for _ in range(3):
        y = y * (1.5 - 0.5 * x * y * y)
    return y


def _sqrt(x):
    xs = jnp.maximum(x, 1e-30)
    return xs * _rsqrt(xs)


def _body(n_chunks, emb,
          up_hbm, uc_hbm, ip_hbm, ic_hbm, ib_hbm,
          utab, itab, bias1d,
          pred_out, ul2_out, il2_out,
          upv, ucv, ipv, icv, ibv,
          ub0, it0, bb0, ub1, it1, bb1,
          pred_v, ul2_v, il2_v, sem0, sem1):
    wid = lax.axis_index("s") * NC + lax.axis_index("c")
    p_sub = n_chunks * CHUNK
    base = wid * p_sub

    pltpu.sync_copy(up_hbm.at[wid], upv)
    pltpu.sync_copy(uc_hbm.at[wid], ucv)
    pltpu.sync_copy(ip_hbm.at[wid], ipv)
    pltpu.sync_copy(ic_hbm.at[wid], icv)
    pltpu.sync_copy(ib_hbm.at[wid], ibv)

    bufs = [(ub0, it0, bb0, sem0), (ub1, it1, bb1, sem1)]

    def fire(j, ub, it, bb, sem):
        pltpu.async_copy(utab.at[upv.at[j]], ub, sem)
        pltpu.async_copy(itab.at[ipv.at[j]], it, sem)
        pltpu.async_copy(bias1d.at[ibv.at[j]], bb, sem)

    def drain(ub, it, bb, sem):
        pltpu.make_async_copy(utab.at[pl.ds(0, CHUNK)], ub, sem).wait()
        pltpu.make_async_copy(itab.at[pl.ds(0, CHUNK)], it, sem).wait()
        pltpu.make_async_copy(bias1d.at[pl.ds(0, CHUNK)], bb, sem).wait()

    iota = lax.iota(jnp.int32, L)

    fire(0, *bufs[0])
    for j in range(n_chunks):
        ub, it, bb, sem = bufs[j % 2]
        if j + 1 < n_chunks:
            fire(j + 1, *bufs[(j + 1) % 2])
        drain(ub, it, bb, sem)

        def group(g, _, j=j, ub=ub, it=it, bb=bb):
            slot = iota + g * L
            ucol = ucv[j, pl.ds(g * L, L)]
            icol = icv[j, pl.ds(g * L, L)]
            acc_d = jnp.zeros((L,), jnp.float32)
            acc_u2 = jnp.zeros((L,), jnp.float32)
            acc_i2 = jnp.zeros((L,), jnp.float32)
            for e in range(emb):
                u_e = plsc.load_gather(ub, [slot, ucol + e])
                i_e = plsc.load_gather(it, [slot, icol + e])
                acc_d = acc_d + u_e * i_e
                acc_u2 = acc_u2 + u_e * u_e
                acc_i2 = acc_i2 + i_e * i_e
            bias = bb[pl.ds(g * L, L)]
            norm_u = _sqrt(acc_u2)
            norm_i = _sqrt(acc_i2)
            su = jnp.minimum(1.0, MAX_NORM / (norm_u + EPS))
            si = jnp.minimum(1.0, MAX_NORM / (norm_i + EPS))
            sl = pl.ds(j * CHUNK + g * L, L)
            pred_v[sl] = acc_d * (su * si) + bias
            ul2_v[sl] = norm_u * su
            il2_v[sl] = _sqrt(acc_i2 * (si * si) + bias * bias)
            return 0

        lax.fori_loop(0, CHUNK // L, group, 0)

    out_sl = pl.ds(base, p_sub)
    pltpu.sync_copy(pred_v, pred_out.at[out_sl])
    pltpu.sync_copy(ul2_v, ul2_out.at[out_sl])
    pltpu.sync_copy(il2_v, il2_out.at[out_sl])


def kernel(user_idx, item_i_idx, user_table, item_table, item_bias_table):
    b = user_idx.shape[0]
    n_tab, emb = user_table.shape
    p_sub = b // NW
    n_chunks = p_sub // CHUNK
    rps = 128 // emb
    sh = {4: 2, 2: 1, 1: 0}[rps]

    uidx = user_idx.astype(jnp.int32)
    iidx = item_i_idx.astype(jnp.int32)
    shape3 = (NW, n_chunks, CHUNK)
    up3 = (uidx >> sh).reshape(shape3)
    uc3 = ((uidx & (rps - 1)) * emb).reshape(shape3)
    ip3 = (iidx >> sh).reshape(shape3)
    ic3 = ((iidx & (rps - 1)) * emb).reshape(shape3)
    ib3 = iidx.reshape(shape3)

    utab = user_table.reshape(n_tab // rps, 128)
    itab = item_table.reshape(n_tab // rps, 128)
    bias1d = item_bias_table.reshape(-1)

    mesh = plsc.VectorSubcoreMesh(
        core_axis_name="c", subcore_axis_name="s",
        num_cores=NC, num_subcores=NS)

    f32 = jnp.float32
    i32 = jnp.int32
    out = pl.kernel(
        functools.partial(_body, n_chunks, emb),
        out_type=[jax.ShapeDtypeStruct((b,), f32)] * 3,
        mesh=mesh,
        compiler_params=pltpu.CompilerParams(needs_layout_passes=False),
        scratch_types=[
            pltpu.VMEM((n_chunks, CHUNK), i32),
            pltpu.VMEM((n_chunks, CHUNK), i32),
            pltpu.VMEM((n_chunks, CHUNK), i32),
            pltpu.VMEM((n_chunks, CHUNK), i32),
            pltpu.VMEM((n_chunks, CHUNK), i32),
            pltpu.VMEM((CHUNK, 128), f32),
            pltpu.VMEM((CHUNK, 128), f32),
            pltpu.VMEM((CHUNK,), f32),
            pltpu.VMEM((CHUNK, 128), f32),
            pltpu.VMEM((CHUNK, 128), f32),
            pltpu.VMEM((CHUNK,), f32),
            pltpu.VMEM((p_sub,), f32),
            pltpu.VMEM((p_sub,), f32),
            pltpu.VMEM((p_sub,), f32),
            pltpu.SemaphoreType.DMA,
            pltpu.SemaphoreType.DMA,
        ],
    )(up3, uc3, ip3, ic3, ib3, utab, itab, bias1d)
    pred, ul2, il2 = out
    return pred.reshape(b, 1), ul2, il2


# speedup vs baseline: 1.0973x; 1.0973x over previous
import functools

import jax
import jax.numpy as jnp
from jax import lax
from jax.experimental import pallas as pl
from jax.experimental.pallas import tpu as pltpu
from jax.experimental.pallas import tpu_sc as plsc

NC = 2
NS = 16
NW = NC * NS
L = 16

MAX_NORM = 1.0
EPS = 1e-7


def _rsqrt(x):
    i = plsc.bitcast(x, jnp.int32)
    i = jnp.int32(0x5F3759DF) - (i >> 1)
    y = plsc.bitcast(i, jnp.float32)
    for _ in range(3):
        y = y * (1.5 - 0.5 * x * y * y)
    return y


def _sqrt(x):
    xs = jnp.maximum(x, 1e-30)
    return xs * _rsqrt(xs)


def _body(p_sub, emb,
          uidx_hbm, iidx_hbm, utab, itab, btab,
          pred_out, ul2_out, il2_out,
          uidx_v, iidx_v,
          ub0, ib0, bb0, ub1, ib1, bb1,
          pred_v, ul2_v, il2_v, sem0, sem1):
    wid = lax.axis_index("s") * NC + lax.axis_index("c")
    base = wid * p_sub
    n_batch = p_sub // L

    pltpu.sync_copy(uidx_hbm.at[wid], uidx_v)
    pltpu.sync_copy(iidx_hbm.at[wid], iidx_v)

    lane = lax.iota(jnp.int32, L)

    def fire(g, ub, ib, bb, sem):
        uvec = uidx_v[pl.ds(g * L, L)]
        ivec = iidx_v[pl.ds(g * L, L)]
        for rr in range(L):
            iu = jnp.sum(jnp.where(lane == rr, uvec, 0))
            ii = jnp.sum(jnp.where(lane == rr, ivec, 0))
            sl = pl.ds(rr, 1)
            pltpu.async_copy(utab.at[pl.ds(iu, 1)], ub.at[sl], sem)
            pltpu.async_copy(itab.at[pl.ds(ii, 1)], ib.at[sl], sem)
            pltpu.async_copy(btab.at[pl.ds(ii, 1)], bb.at[sl], sem)

    def drain(ub, ib, bb, sem):
        pltpu.make_async_copy(utab.at[pl.ds(0, L)], ub, sem).wait()
        pltpu.make_async_copy(itab.at[pl.ds(0, L)], ib, sem).wait()
        pltpu.make_async_copy(btab.at[pl.ds(0, L)], bb, sem).wait()

    iota = lax.iota(jnp.int32, L)
    zeros = jnp.zeros((L,), jnp.int32)

    def compute(g, ub, ib, bb):
        acc_d = jnp.zeros((L,), jnp.float32)
        acc_u2 = jnp.zeros((L,), jnp.float32)
        acc_i2 = jnp.zeros((L,), jnp.float32)
        for e in range(emb):
            ecol = jnp.full((L,), e, jnp.int32)
            u_e = plsc.load_gather(ub, [iota, ecol])
            i_e = plsc.load_gather(ib, [iota, ecol])
            acc_d = acc_d + u_e * i_e
            acc_u2 = acc_u2 + u_e * u_e
            acc_i2 = acc_i2 + i_e * i_e
        bias = plsc.load_gather(bb, [iota, zeros])
        norm_u = _sqrt(acc_u2)
        norm_i = _sqrt(acc_i2)
        su = jnp.minimum(1.0, MAX_NORM / (norm_u + EPS))
        si = jnp.minimum(1.0, MAX_NORM / (norm_i + EPS))
        sl = pl.ds(g * L, L)
        pred_v[sl] = acc_d * (su * si) + bias
        ul2_v[sl] = norm_u * su
        il2_v[sl] = _sqrt(acc_i2 * (si * si) + bias * bias)

    fire(0, ub0, ib0, bb0, sem0)

    def step(gg, _):
        g0 = 2 * gg
        fire(g0 + 1, ub1, ib1, bb1, sem1)
        drain(ub0, ib0, bb0, sem0)
        compute(g0, ub0, ib0, bb0)

        @pl.when(gg < n_batch // 2 - 1)
        def _():
            fire(g0 + 2, ub0, ib0, bb0, sem0)

        drain(ub1, ib1, bb1, sem1)
        compute(g0 + 1, ub1, ib1, bb1)
        return 0

    lax.fori_loop(0, n_batch // 2, step, 0)

    out_sl = pl.ds(base, p_sub)
    pltpu.sync_copy(pred_v, pred_out.at[out_sl])
    pltpu.sync_copy(ul2_v, ul2_out.at[out_sl])
    pltpu.sync_copy(il2_v, il2_out.at[out_sl])


def kernel(user_idx, item_i_idx, user_table, item_table, item_bias_table):
    b = user_idx.shape[0]
    emb = user_table.shape[1]
    p_sub = b // NW

    uidx2 = user_idx.astype(jnp.int32).reshape(NW, p_sub)
    iidx2 = item_i_idx.astype(jnp.int32).reshape(NW, p_sub)

    mesh = plsc.VectorSubcoreMesh(
        core_axis_name="c", subcore_axis_name="s",
        num_cores=NC, num_subcores=NS)

    f32 = jnp.float32
    i32 = jnp.int32
    out = pl.kernel(
        functools.partial(_body, p_sub, emb),
        out_type=[jax.ShapeDtypeStruct((b,), f32)] * 3,
        mesh=mesh,
        compiler_params=pltpu.CompilerParams(needs_layout_passes=False),
        scratch_types=[
            pltpu.VMEM((p_sub,), i32),
            pltpu.VMEM((p_sub,), i32),
            pltpu.VMEM((L, emb), f32),
            pltpu.VMEM((L, emb), f32),
            pltpu.VMEM((L, 1), f32),
            pltpu.VMEM((L, emb), f32),
            pltpu.VMEM((L, emb), f32),
            pltpu.VMEM((L, 1), f32),
            pltpu.VMEM((p_sub,), f32),
            pltpu.VMEM((p_sub,), f32),
            pltpu.VMEM((p_sub,), f32),
            pltpu.SemaphoreType.DMA,
            pltpu.SemaphoreType.DMA,
        ],
    )(uidx2, iidx2, user_table, item_table, item_bias_table)
    pred, ul2, il2 = out
    return pred.reshape(b, 1), ul2, il2
